# baseline (speedup 1.0000x reference)
import functools

import jax
import jax.numpy as jnp
import numpy as np
from jax import lax
from jax.experimental import pallas as pl
from jax.experimental.pallas import tpu as pltpu
from jax.experimental.pallas import tpu_sc as plsc

_T = 32
_TAU = 10.0
_EPS = 1e-7
_B = 1024
_I = 1024
_NW = 32
_ROWS = _B // _NW
_GRP = _I // 16

_j = np.arange(1, _T, dtype=np.float64)
_s = np.exp(-_j / _TAU) - _EPS
_THR = np.concatenate(
    [[np.inf], np.log(_s / (1.0 - _s))]
).astype(np.float32)


def _worker_id():
    return lax.axis_index("s") * 2 + lax.axis_index("c")


def _sc_body(x_hbm, zeros_hbm, thr_hbm, out_hbm,
             xblk, thr_v, buf0, buf1, idx0, idx1, sem0, sem1):
    base = _worker_id() * _ROWS
    pltpu.sync_copy(thr_hbm, thr_v)
    pltpu.sync_copy(x_hbm.at[pl.ds(base, _ROWS)], xblk)
    pltpu.async_copy(zeros_hbm, buf0, sem0)
    pltpu.async_copy(zeros_hbm, buf1, sem1)

    lane = lax.iota(jnp.int32, 16)
    ones16 = jnp.full((16,), 1.0, jnp.float32)
    zeros16 = jnp.zeros((16,), jnp.float32)
    zeros16i = jnp.zeros((16,), jnp.int32)

    def init_idx(g, _):
        idx0[g, :] = zeros16i
        idx1[g, :] = zeros16i
        return 0

    lax.fori_loop(0, _GRP, init_idx, 0, unroll=8)

    def do_row(r, buf, idx, sem):
        b = base + r
        pltpu.make_async_copy(zeros_hbm, buf, sem).wait()

        def group(g, _):
            i_vec = g * 16 + lane
            t_prev = idx[g, :]
            plsc.store_scatter(buf, [t_prev, i_vec], zeros16)
            xv = xblk[r, pl.ds(g * 16, 16)]
            t = jnp.zeros((16,), jnp.int32)
            for step in (16, 8, 4, 2, 1):
                cand = t + step
                thr_c = plsc.load_gather(thr_v, [cand])
                t = jnp.where(xv <= thr_c, cand, t)
            idx[g, :] = t
            plsc.store_scatter(buf, [t, i_vec], ones16)
            return 0

        lax.fori_loop(0, _GRP, group, 0, unroll=4)
        pltpu.async_copy(buf, out_hbm.at[b], sem)

    def pair(k, _):
        do_row(2 * k, buf0, idx0, sem0)
        do_row(2 * k + 1, buf1, idx1, sem1)
        return 0

    lax.fori_loop(0, _ROWS // 2, pair, 0)
    pltpu.make_async_copy(zeros_hbm, buf0, sem0).wait()
    pltpu.make_async_copy(zeros_hbm, buf1, sem1).wait()


@jax.jit
def kernel(x):
    mesh = plsc.VectorSubcoreMesh(core_axis_name="c", subcore_axis_name="s")
    sc = functools.partial(
        pl.kernel,
        mesh=mesh,
        out_type=jax.ShapeDtypeStruct((_B, _T, _I), jnp.float32),
        scratch_types=[
            pltpu.VMEM((_ROWS, _I), jnp.float32),
            pltpu.VMEM((_T,), jnp.float32),
            pltpu.VMEM((_T, _I), jnp.float32),
            pltpu.VMEM((_T, _I), jnp.float32),
            pltpu.VMEM((_GRP, 16), jnp.int32),
            pltpu.VMEM((_GRP, 16), jnp.int32),
            pltpu.SemaphoreType.DMA,
            pltpu.SemaphoreType.DMA,
        ],
        compiler_params=pltpu.CompilerParams(
            use_tc_tiling_on_sc=False, needs_layout_passes=False
        ),
    )(_sc_body)
    zeros_plane = jnp.zeros((_T, _I), jnp.float32)
    thr = jnp.asarray(_THR)
    return sc(x, zeros_plane, thr)


# speedup vs baseline: 2.0722x; 2.0722x over previous
import functools

import jax
import jax.numpy as jnp
import numpy as np
from jax import lax
from jax.experimental import pallas as pl
from jax.experimental.pallas import tpu as pltpu
from jax.experimental.pallas import tpu_sc as plsc

_T = 32
_TAU = 10.0
_EPS = 1e-7
_B = 1024
_I = 1024
_NW = 32
_ROWS = _B // _NW
_GRP = _I // 16

_j = np.arange(1, _T, dtype=np.float64)
_s = np.exp(-_j / _TAU) - _EPS
_THR = np.concatenate(
    [[np.inf], np.log(_s / (1.0 - _s))]
).astype(np.float32)


def _worker_id():
    return lax.axis_index("s") * 2 + lax.axis_index("c")


def _sc_body(x_hbm, zeros_hbm, thr_hbm, out_hbm,
             xblk, thr_v, buf0, buf1, idx0, idx1, sem0, sem1):
    base = _worker_id() * _ROWS
    pltpu.sync_copy(thr_hbm, thr_v)
    pltpu.sync_copy(x_hbm.at[pl.ds(base, _ROWS)], xblk)
    pltpu.async_copy(zeros_hbm, buf0, sem0)
    pltpu.async_copy(zeros_hbm, buf1, sem1)

    lane = lax.iota(jnp.int32, 16)
    ones16 = jnp.full((16,), 1.0, jnp.float32)
    zeros16 = jnp.zeros((16,), jnp.float32)
    zeros16i = jnp.zeros((16,), jnp.int32)

    def init_idx(g, _):
        idx0[pl.ds(g * 16, 16)] = zeros16i
        idx1[pl.ds(g * 16, 16)] = zeros16i
        return 0

    lax.fori_loop(0, _GRP, init_idx, 0, unroll=8)

    def do_row(r, buf, idx, sem):
        b = base + r
        pltpu.make_async_copy(zeros_hbm, buf, sem).wait()

        def group(g, _):
            i_vec = g * 16 + lane
            t_prev = idx[pl.ds(g * 16, 16)]
            plsc.store_scatter(buf, [t_prev, i_vec], zeros16)
            xv = xblk[r, pl.ds(g * 16, 16)]
            t = jnp.zeros((16,), jnp.int32)
            for step in (16, 8, 4, 2, 1):
                cand = t + step
                thr_c = plsc.load_gather(thr_v, [cand])
                t = jnp.where(xv <= thr_c, cand, t)
            idx[pl.ds(g * 16, 16)] = t
            plsc.store_scatter(buf, [t, i_vec], ones16)
            return 0

        lax.fori_loop(0, _GRP, group, 0, unroll=4)
        pltpu.async_copy(buf, out_hbm.at[b], sem)

    def pair(k, _):
        do_row(2 * k, buf0, idx0, sem0)
        do_row(2 * k + 1, buf1, idx1, sem1)
        return 0

    lax.fori_loop(0, _ROWS // 2, pair, 0)
    pltpu.make_async_copy(zeros_hbm, buf0, sem0).wait()
    pltpu.make_async_copy(zeros_hbm, buf1, sem1).wait()


@jax.jit
def kernel(x):
    mesh = plsc.VectorSubcoreMesh(core_axis_name="c", subcore_axis_name="s")
    sc = functools.partial(
        pl.kernel,
        mesh=mesh,
        out_type=jax.ShapeDtypeStruct((_B, _T, _I), jnp.float32),
        scratch_types=[
            pltpu.VMEM((_ROWS, _I), jnp.float32),
            pltpu.VMEM((_T,), jnp.float32),
            pltpu.VMEM((_T, _I), jnp.float32),
            pltpu.VMEM((_T, _I), jnp.float32),
            pltpu.VMEM((_GRP * 16,), jnp.int32),
            pltpu.VMEM((_GRP * 16,), jnp.int32),
            pltpu.SemaphoreType.DMA,
            pltpu.SemaphoreType.DMA,
        ],
        compiler_params=pltpu.CompilerParams(
            use_tc_tiling_on_sc=True, needs_layout_passes=False
        ),
    )(_sc_body)
    zeros_plane = jnp.zeros((_T, _I), jnp.float32)
    thr = jnp.asarray(_THR)
    return sc(x, zeros_plane, thr)


# speedup vs baseline: 2.4980x; 1.2055x over previous
import functools

import jax
import jax.numpy as jnp
import numpy as np
from jax import lax
from jax.experimental import pallas as pl
from jax.experimental.pallas import tpu as pltpu
from jax.experimental.pallas import tpu_sc as plsc

_T = 32
_TAU = 10.0
_EPS = 1e-7
_B = 1024
_I = 1024
_NW = 32
_ROWS = _B // _NW
_GRP = _I // 16

_j = np.arange(1, _T, dtype=np.float64)
_s = np.exp(-_j / _TAU) - _EPS
_THR = np.concatenate(
    [[np.inf], np.log(_s / (1.0 - _s))]
).astype(np.float32)


def _worker_id():
    return lax.axis_index("s") * 2 + lax.axis_index("c")


def _sc_body(x_hbm, zeros_hbm, out_hbm,
             xblk, buf0, buf1, idx0, idx1, sem0, sem1):
    base = _worker_id() * _ROWS
    pltpu.sync_copy(x_hbm.at[pl.ds(base, _ROWS)], xblk)
    pltpu.async_copy(zeros_hbm, buf0, sem0)
    pltpu.async_copy(zeros_hbm, buf1, sem1)

    lane = lax.iota(jnp.int32, 16)
    ones16 = jnp.full((16,), 1.0, jnp.float32)
    zeros16 = jnp.zeros((16,), jnp.float32)
    zeros16i = jnp.zeros((16,), jnp.int32)

    def init_idx(g, _):
        idx0[pl.ds(g * 16, 16)] = zeros16i
        idx1[pl.ds(g * 16, 16)] = zeros16i
        return 0

    lax.fori_loop(0, _GRP, init_idx, 0, unroll=8)

    def do_row(r, buf, idx, sem):
        b = base + r
        pltpu.make_async_copy(zeros_hbm, buf, sem).wait()

        def group(g, _):
            i_vec = g * 16 + lane
            t_prev = idx[pl.ds(g * 16, 16)]
            plsc.store_scatter(buf, [t_prev, i_vec], zeros16)
            xv = xblk[r, pl.ds(g * 16, 16)]
            terms = [
                (xv <= _THR[j]).astype(jnp.int32) for j in range(1, _T)
            ]
            while len(terms) > 1:
                terms = [
                    terms[k] + terms[k + 1] if k + 1 < len(terms) else terms[k]
                    for k in range(0, len(terms), 2)
                ]
            t = terms[0]
            idx[pl.ds(g * 16, 16)] = t
            plsc.store_scatter(buf, [t, i_vec], ones16)
            return 0

        lax.fori_loop(0, _GRP, group, 0, unroll=8)
        pltpu.async_copy(buf, out_hbm.at[b], sem)

    def pair(k, _):
        do_row(2 * k, buf0, idx0, sem0)
        do_row(2 * k + 1, buf1, idx1, sem1)
        return 0

    lax.fori_loop(0, _ROWS // 2, pair, 0)
    pltpu.make_async_copy(zeros_hbm, buf0, sem0).wait()
    pltpu.make_async_copy(zeros_hbm, buf1, sem1).wait()


@jax.jit
def kernel(x):
    mesh = plsc.VectorSubcoreMesh(core_axis_name="c", subcore_axis_name="s")
    sc = functools.partial(
        pl.kernel,
        mesh=mesh,
        out_type=jax.ShapeDtypeStruct((_B, _T, _I), jnp.float32),
        scratch_types=[
            pltpu.VMEM((_ROWS, _I), jnp.float32),
            pltpu.VMEM((_T, _I), jnp.float32),
            pltpu.VMEM((_T, _I), jnp.float32),
            pltpu.VMEM((_GRP * 16,), jnp.int32),
            pltpu.VMEM((_GRP * 16,), jnp.int32),
            pltpu.SemaphoreType.DMA,
            pltpu.SemaphoreType.DMA,
        ],
        compiler_params=pltpu.CompilerParams(
            use_tc_tiling_on_sc=True, needs_layout_passes=False
        ),
    )(_sc_body)
    zeros_plane = jnp.zeros((_T, _I), jnp.float32)
    return sc(x, zeros_plane)


# speedup vs baseline: 2.6798x; 1.0728x over previous
import functools

import jax
import jax.numpy as jnp
import numpy as np
from jax import lax
from jax.experimental import pallas as pl
from jax.experimental.pallas import tpu as pltpu
from jax.experimental.pallas import tpu_sc as plsc

_T = 32
_TAU = 10.0
_EPS = 1e-7
_B = 1024
_I = 1024
_NW = 32
_ROWS = _B // _NW
_GRP = _I // 16

_jj = np.arange(1, _T, dtype=np.float64)
_ss = np.exp(-_jj / _TAU) - _EPS
_D = np.log(_ss / (1.0 - _ss)).astype(np.float32)

_LO = np.float32(-3.25)
_INVH = np.float32(16.0)
_N = 96
_cells = [int(np.trunc((d - _LO) * _INVH)) for d in _D]
assert len(set(_cells)) == len(_cells) and min(_cells) > 0 and max(_cells) < _N - 1
_A = np.zeros(_N, np.int32)
_R = np.full(_N, np.inf, np.float32)
for _c in range(_N):
    _cnt = sum(1 for cc in _cells if cc > _c)
    if _c in _cells:
        _A[_c] = _cnt
        _R[_c] = _D[_cells.index(_c)]
    else:
        _A[_c] = _cnt - 1


def _worker_id():
    return lax.axis_index("s") * 2 + lax.axis_index("c")


def _sc_body(x_hbm, zeros_hbm, a_hbm, r_hbm, out_hbm,
             xblk, a_v, r_v, buf0, buf1, idx0, idx1, sem0, sem1):
    base = _worker_id() * _ROWS
    pltpu.sync_copy(a_hbm, a_v)
    pltpu.sync_copy(r_hbm, r_v)
    pltpu.sync_copy(x_hbm.at[pl.ds(base, _ROWS)], xblk)
    pltpu.async_copy(zeros_hbm, buf0, sem0)
    pltpu.async_copy(zeros_hbm, buf1, sem1)

    lane = lax.iota(jnp.int32, 16)
    ones16 = jnp.full((16,), 1.0, jnp.float32)
    zeros16 = jnp.zeros((16,), jnp.float32)
    zeros16i = jnp.zeros((16,), jnp.int32)

    def init_idx(g, _):
        idx0[pl.ds(g * 16, 16)] = zeros16i
        idx1[pl.ds(g * 16, 16)] = zeros16i
        return 0

    lax.fori_loop(0, _GRP, init_idx, 0, unroll=8)

    def do_row(r, buf, idx, sem):
        b = base + r
        pltpu.make_async_copy(zeros_hbm, buf, sem).wait()

        def group(g, _):
            i_vec = g * 16 + lane
            t_prev = idx[pl.ds(g * 16, 16)]
            plsc.store_scatter(buf, [t_prev, i_vec], zeros16)
            xv = xblk[r, pl.ds(g * 16, 16)]
            ci = ((xv - _LO) * _INVH).astype(jnp.int32)
            ci = jnp.minimum(jnp.maximum(ci, 0), _N - 1)
            a = plsc.load_gather(a_v, [ci])
            rr = plsc.load_gather(r_v, [ci])
            t = a + (xv <= rr).astype(jnp.int32)
            idx[pl.ds(g * 16, 16)] = t
            plsc.store_scatter(buf, [t, i_vec], ones16)
            return 0

        lax.fori_loop(0, _GRP, group, 0, unroll=8)
        pltpu.async_copy(buf, out_hbm.at[b], sem)

    def pair(k, _):
        do_row(2 * k, buf0, idx0, sem0)
        do_row(2 * k + 1, buf1, idx1, sem1)
        return 0

    lax.fori_loop(0, _ROWS // 2, pair, 0)
    pltpu.make_async_copy(zeros_hbm, buf0, sem0).wait()
    pltpu.make_async_copy(zeros_hbm, buf1, sem1).wait()


@jax.jit
def kernel(x):
    mesh = plsc.VectorSubcoreMesh(core_axis_name="c", subcore_axis_name="s")
    sc = functools.partial(
        pl.kernel,
        mesh=mesh,
        out_type=jax.ShapeDtypeStruct((_B, _T, _I), jnp.float32),
        scratch_types=[
            pltpu.VMEM((_ROWS, _I), jnp.float32),
            pltpu.VMEM((_N,), jnp.int32),
            pltpu.VMEM((_N,), jnp.float32),
            pltpu.VMEM((_T, _I), jnp.float32),
            pltpu.VMEM((_T, _I), jnp.float32),
            pltpu.VMEM((_GRP * 16,), jnp.int32),
            pltpu.VMEM((_GRP * 16,), jnp.int32),
            pltpu.SemaphoreType.DMA,
            pltpu.SemaphoreType.DMA,
        ],
        compiler_params=pltpu.CompilerParams(
            use_tc_tiling_on_sc=True, needs_layout_passes=False
        ),
    )(_sc_body)
    zeros_plane = jnp.zeros((_T, _I), jnp.float32)
    return sc(x, zeros_plane, jnp.asarray(_A), jnp.asarray(_R))


# speedup vs baseline: 2.6873x; 1.0028x over previous
import functools

import jax
import jax.numpy as jnp
import numpy as np
from jax import lax
from jax.experimental import pallas as pl
from jax.experimental.pallas import tpu as pltpu
from jax.experimental.pallas import tpu_sc as plsc

_T = 32
_TAU = 10.0
_EPS = 1e-7
_B = 1024
_I = 1024
_NW = 32
_ROWS = _B // _NW
_GRP = _I // 16

_jj = np.arange(1, _T, dtype=np.float64)
_ss = np.exp(-_jj / _TAU) - _EPS
_D = np.log(_ss / (1.0 - _ss)).astype(np.float32)

_LO = np.float32(-3.25)
_INVH = np.float32(16.0)
_N = 128
_cells = [int(np.trunc((d - _LO) * _INVH)) for d in _D]
assert len(set(_cells)) == len(_cells) and min(_cells) > 0 and max(_cells) < _N - 1
_A = np.zeros(_N, np.int32)
_R = np.full(_N, np.inf, np.float32)
for _c in range(_N):
    _cnt = sum(1 for cc in _cells if cc > _c)
    if _c in _cells:
        _A[_c] = _cnt
        _R[_c] = _D[_cells.index(_c)]
    else:
        _A[_c] = _cnt - 1


def _worker_id():
    return lax.axis_index("s") * 2 + lax.axis_index("c")


def _sc_body(x_hbm, zeros_hbm, a_hbm, r_hbm, out_hbm,
             xblk, a_v, r_v, buf0, buf1, idx0, idx1, sem0, sem1):
    base = _worker_id() * _ROWS
    pltpu.sync_copy(a_hbm, a_v)
    pltpu.sync_copy(r_hbm, r_v)
    pltpu.sync_copy(x_hbm.at[pl.ds(base, _ROWS)], xblk)
    pltpu.async_copy(zeros_hbm, buf0, sem0)
    pltpu.async_copy(zeros_hbm, buf1, sem1)

    lane = lax.iota(jnp.int32, 16)
    ones16 = jnp.full((16,), 1.0, jnp.float32)
    zeros16 = jnp.zeros((16,), jnp.float32)
    zeros16i = jnp.zeros((16,), jnp.int32)

    def init_idx(g, _):
        idx0[pl.ds(g * 16, 16)] = zeros16i
        idx1[pl.ds(g * 16, 16)] = zeros16i
        return 0

    lax.fori_loop(0, _GRP, init_idx, 0, unroll=8)

    def do_row(r, buf, idx, sem):
        b = base + r
        pltpu.make_async_copy(zeros_hbm, buf, sem).wait()

        def group(g, _):
            i_vec = g * 16 + lane
            t_prev = idx[pl.ds(g * 16, 16)]
            plsc.store_scatter(buf, [t_prev, i_vec], zeros16)
            xv = xblk[r, pl.ds(g * 16, 16)]
            ci = ((xv - _LO) * _INVH).astype(jnp.int32)
            ci = jnp.minimum(jnp.maximum(ci, 0), _N - 1)
            a = plsc.load_gather(a_v, [ci])
            rr = plsc.load_gather(r_v, [ci])
            t = a + (xv <= rr).astype(jnp.int32)
            idx[pl.ds(g * 16, 16)] = t
            plsc.store_scatter(buf, [t, i_vec], ones16)
            return 0

        lax.fori_loop(0, _GRP, group, 0, unroll=8)
        pltpu.async_copy(buf, out_hbm.at[b], sem)

    def pair(k, _):
        do_row(2 * k, buf0, idx0, sem0)
        do_row(2 * k + 1, buf1, idx1, sem1)
        return 0

    lax.fori_loop(0, _ROWS // 2, pair, 0)
    pltpu.make_async_copy(zeros_hbm, buf0, sem0).wait()
    pltpu.make_async_copy(zeros_hbm, buf1, sem1).wait()


@jax.jit
def kernel(x):
    mesh = plsc.VectorSubcoreMesh(core_axis_name="c", subcore_axis_name="s")
    sc = functools.partial(
        pl.kernel,
        mesh=mesh,
        out_type=jax.ShapeDtypeStruct((_B, _T, _I), jnp.float32),
        scratch_types=[
            pltpu.VMEM((_ROWS, _I), jnp.float32),
            pltpu.VMEM((_N,), jnp.int32),
            pltpu.VMEM((_N,), jnp.float32),
            pltpu.VMEM((_T, _I), jnp.float32),
            pltpu.VMEM((_T, _I), jnp.float32),
            pltpu.VMEM((_GRP * 16,), jnp.int32),
            pltpu.VMEM((_GRP * 16,), jnp.int32),
            pltpu.SemaphoreType.DMA,
            pltpu.SemaphoreType.DMA,
        ],
        compiler_params=pltpu.CompilerParams(
            use_tc_tiling_on_sc=True, needs_layout_passes=False
        ),
    )(_sc_body)
    zeros_plane = jnp.zeros((_T, _I), jnp.float32)
    return sc(x, zeros_plane, jnp.asarray(_A), jnp.asarray(_R))
